# baseline (speedup 1.0000x reference)
import functools

import jax
import jax.numpy as jnp
from jax import lax
from jax.experimental import pallas as pl
from jax.experimental.pallas import tpu as pltpu
from jax.experimental.pallas import tpu_sc as plsc

_D = 64
_B = 4096
_K = 20
_J = 5
_NC, _NS = 2, 16
_NW = _NC * _NS
_BPW = _B // _NW
_CB = 4
_NCH = _BPW // _CB
_PAIRS = _CB * _K
_GROUPS = _PAIRS // 16
_NROWS = _CB * _J * _K
_TROW = 17

_mesh = plsc.VectorSubcoreMesh(core_axis_name="c", subcore_axis_name="s")


def _chunk_buffers():
  return [
      pltpu.VMEM((_PAIRS,), jnp.int32),
      pltpu.VMEM((_PAIRS,), jnp.int32),
      pltpu.VMEM((_CB, _J * _K), jnp.int32),
      pltpu.VMEM((_PAIRS, _D), jnp.float32),
      pltpu.VMEM((_PAIRS, _D), jnp.float32),
      pltpu.VMEM((_NROWS, _D), jnp.float32),
      pltpu.VMEM((_PAIRS,), jnp.float32),
      pltpu.VMEM((_NROWS,), jnp.float32),
      pltpu.SemaphoreType.DMA,
      pltpu.SemaphoreType.DMA,
  ]


@functools.partial(
    pl.kernel,
    out_type=(
        jax.ShapeDtypeStruct((_B * _K,), jnp.float32),
        jax.ShapeDtypeStruct((_B * _J * _K,), jnp.float32),
    ),
    mesh=_mesh,
    compiler_params=pltpu.CompilerParams(
        needs_layout_passes=False, use_tc_tiling_on_sc=False),
    scratch_types=_chunk_buffers() + _chunk_buffers() + [
        pltpu.VMEM((96 * _TROW,), jnp.float32),
    ],
)
def _sgns(c_hbm, o_hbm, neg_hbm, v_hbm, u_hbm, pos_hbm, nout_hbm, *bufs):
  wid = lax.axis_index("s") * _NC + lax.axis_index("c")
  iota = lax.broadcasted_iota(jnp.int32, (16,), 0)
  sets = (bufs[:10], bufs[10:20])
  tmp = bufs[20]

  def idx_slices(ch):
    b0 = wid * _BPW + ch * _CB
    p0 = b0 * _K
    return (c_hbm.at[pl.ds(p0, _PAIRS)], o_hbm.at[pl.ds(p0, _PAIRS)],
            neg_hbm.at[pl.ds(b0, _CB)])

  def stage_idx(s, ch):
    cidx, oidx, nidx, sem = s[0], s[1], s[2], s[8]
    csl, osl, nsl = idx_slices(ch)
    pltpu.async_copy(csl, cidx, sem)
    pltpu.async_copy(osl, oidx, sem)
    pltpu.async_copy(nsl, nidx, sem)

  def wait_idx(s, ch):
    cidx, oidx, nidx, sem = s[0], s[1], s[2], s[8]
    csl, osl, nsl = idx_slices(ch)
    pltpu.make_async_copy(csl, cidx, sem).wait()
    pltpu.make_async_copy(osl, oidx, sem).wait()
    pltpu.make_async_copy(nsl, nidx, sem).wait()

  def fire_gathers(s):
    cidx, oidx, nidx, vbuf, ubuf, nbuf, sem = (
        s[0], s[1], s[2], s[3], s[4], s[5], s[9])
    pltpu.async_copy(v_hbm.at[cidx], vbuf, sem)
    pltpu.async_copy(u_hbm.at[oidx], ubuf, sem)
    for i in range(_CB):
      pltpu.async_copy(u_hbm.at[nidx.at[i]],
                       nbuf.at[pl.ds(i * _J * _K, _J * _K)], sem)

  def wait_gathers(s):
    cidx, oidx, nidx, vbuf, ubuf, nbuf, sem = (
        s[0], s[1], s[2], s[3], s[4], s[5], s[9])
    pltpu.make_async_copy(v_hbm.at[cidx], vbuf, sem).wait()
    pltpu.make_async_copy(u_hbm.at[oidx], ubuf, sem).wait()
    for i in range(_CB):
      pltpu.make_async_copy(u_hbm.at[nidx.at[i]],
                            nbuf.at[pl.ds(i * _J * _K, _J * _K)], sem).wait()

  def compute(s, ch):
    vbuf, ubuf, nbuf, posbuf, noutbuf = s[3], s[4], s[5], s[6], s[7]
    b0 = wid * _BPW + ch * _CB
    one = jnp.float32(1.0)
    lane15 = iota * _TROW + 15

    def group_body(g, carry):
      pg = g * 16
      pvec = pg + iota
      bbv = lax.div(pvec, _K)
      nr0 = bbv * (_J * _K) + (pvec - bbv * _K)

      for i in range(16):
        p = pg + i
        bb = lax.div(p, _K)
        kk = p - bb * _K
        nbase = bb * (_J * _K) + kk
        vv = [vbuf[p, pl.ds(16 * t, 16)] for t in range(4)]
        uu = [ubuf[p, pl.ds(16 * t, 16)] for t in range(4)]
        sp = (vv[0] * uu[0] + vv[1] * uu[1]
              + vv[2] * uu[2] + vv[3] * uu[3])
        tmp[pl.ds(i * _TROW, 16)] = plsc.cumsum(sp)
        for j in range(_J):
          nr = nbase + j * _K
          nn = [nbuf[nr, pl.ds(16 * t, 16)] for t in range(4)]
          sn = (vv[0] * nn[0] + vv[1] * nn[1]
                + vv[2] * nn[2] + vv[3] * nn[3])
          tmp[pl.ds((16 + j * 16 + i) * _TROW, 16)] = plsc.cumsum(sn)

      posv = plsc.load_gather(tmp, [lane15])
      posbuf[pl.ds(pg, 16)] = one / (one + jnp.exp(-posv))
      for j in range(_J):
        nv = plsc.load_gather(tmp, [(16 + j * 16) * _TROW + lane15])
        plsc.store_scatter(noutbuf, [nr0 + j * _K],
                           one / (one + jnp.exp(nv)))
      return carry

    lax.fori_loop(0, _GROUPS, group_body, 0)
    pltpu.sync_copy(posbuf, pos_hbm.at[pl.ds(b0 * _K, _PAIRS)])
    pltpu.sync_copy(noutbuf, nout_hbm.at[pl.ds(b0 * (_J * _K), _NROWS)])

  stage_idx(sets[0], 0)
  wait_idx(sets[0], 0)
  fire_gathers(sets[0])
  stage_idx(sets[1], 1)

  def body(i, carry):
    e = 2 * i
    o = e + 1
    wait_idx(sets[1], o)
    fire_gathers(sets[1])

    wait_gathers(sets[0])

    @pl.when(e + 2 < _NCH)
    def _():
      stage_idx(sets[0], e + 2)

    compute(sets[0], e)

    @pl.when(e + 2 < _NCH)
    def _():
      wait_idx(sets[0], e + 2)
      fire_gathers(sets[0])

    wait_gathers(sets[1])

    @pl.when(o + 2 < _NCH)
    def _():
      stage_idx(sets[1], o + 2)

    compute(sets[1], o)
    return carry

  lax.fori_loop(0, _NCH // 2, body, 0)


def kernel(c, o, neg, vEmb, uEmb):
  c_f = c.reshape(-1).astype(jnp.int32)
  o_f = o.reshape(-1).astype(jnp.int32)
  neg_f = neg.reshape(_B, _J * _K).astype(jnp.int32)
  pos, nout = _sgns(c_f, o_f, neg_f, vEmb, uEmb)
  return pos.reshape(_B, _K), nout.reshape(_B, _J, _K)


# speedup vs baseline: 1.7112x; 1.7112x over previous
import functools

import jax
import jax.numpy as jnp
from jax import lax
from jax.experimental import pallas as pl
from jax.experimental.pallas import tpu as pltpu
from jax.experimental.pallas import tpu_sc as plsc

_D = 64
_B = 4096
_K = 20
_J = 5
_NC, _NS = 2, 16
_NW = _NC * _NS
_BPW = _B // _NW
_CB = 4
_NCH = _BPW // _CB
_PAIRS = _CB * _K
_GROUPS = _PAIRS // 16
_NROWS = _CB * _J * _K

_mesh = plsc.VectorSubcoreMesh(core_axis_name="c", subcore_axis_name="s")


def _chunk_buffers():
  return [
      pltpu.VMEM((_PAIRS,), jnp.int32),
      pltpu.VMEM((_PAIRS,), jnp.int32),
      pltpu.VMEM((_CB, _J * _K), jnp.int32),
      pltpu.VMEM((_PAIRS, _D), jnp.float32),
      pltpu.VMEM((_PAIRS, _D), jnp.float32),
      pltpu.VMEM((_NROWS, _D), jnp.float32),
      pltpu.VMEM((_PAIRS,), jnp.float32),
      pltpu.VMEM((_NROWS,), jnp.float32),
      pltpu.SemaphoreType.DMA,
      pltpu.SemaphoreType.DMA,
  ]


@functools.partial(
    pl.kernel,
    out_type=(
        jax.ShapeDtypeStruct((_B * _K,), jnp.float32),
        jax.ShapeDtypeStruct((_B * _J * _K,), jnp.float32),
    ),
    mesh=_mesh,
    compiler_params=pltpu.CompilerParams(
        needs_layout_passes=False, use_tc_tiling_on_sc=False),
    scratch_types=_chunk_buffers() + _chunk_buffers(),
)
def _sgns(c_hbm, o_hbm, neg_hbm, v_hbm, u_hbm, pos_hbm, nout_hbm, *bufs):
  wid = lax.axis_index("s") * _NC + lax.axis_index("c")
  iota = lax.broadcasted_iota(jnp.int32, (16,), 0)
  sets = (bufs[:10], bufs[10:20])

  def idx_slices(ch):
    b0 = wid * _BPW + ch * _CB
    p0 = b0 * _K
    return (c_hbm.at[pl.ds(p0, _PAIRS)], o_hbm.at[pl.ds(p0, _PAIRS)],
            neg_hbm.at[pl.ds(b0, _CB)])

  def stage_idx(s, ch):
    cidx, oidx, nidx, sem = s[0], s[1], s[2], s[8]
    csl, osl, nsl = idx_slices(ch)
    pltpu.async_copy(csl, cidx, sem)
    pltpu.async_copy(osl, oidx, sem)
    pltpu.async_copy(nsl, nidx, sem)

  def wait_idx(s, ch):
    cidx, oidx, nidx, sem = s[0], s[1], s[2], s[8]
    csl, osl, nsl = idx_slices(ch)
    pltpu.make_async_copy(csl, cidx, sem).wait()
    pltpu.make_async_copy(osl, oidx, sem).wait()
    pltpu.make_async_copy(nsl, nidx, sem).wait()

  def fire_gathers(s):
    cidx, oidx, nidx, vbuf, ubuf, nbuf, sem = (
        s[0], s[1], s[2], s[3], s[4], s[5], s[9])
    pltpu.async_copy(v_hbm.at[cidx], vbuf, sem)
    pltpu.async_copy(u_hbm.at[oidx], ubuf, sem)
    for i in range(_CB):
      pltpu.async_copy(u_hbm.at[nidx.at[i]],
                       nbuf.at[pl.ds(i * _J * _K, _J * _K)], sem)

  def wait_gathers(s):
    cidx, oidx, nidx, vbuf, ubuf, nbuf, sem = (
        s[0], s[1], s[2], s[3], s[4], s[5], s[9])
    pltpu.make_async_copy(v_hbm.at[cidx], vbuf, sem).wait()
    pltpu.make_async_copy(u_hbm.at[oidx], ubuf, sem).wait()
    for i in range(_CB):
      pltpu.make_async_copy(u_hbm.at[nidx.at[i]],
                            nbuf.at[pl.ds(i * _J * _K, _J * _K)], sem).wait()

  def compute(s, ch):
    vbuf, ubuf, nbuf, posbuf, noutbuf = s[3], s[4], s[5], s[6], s[7]
    b0 = wid * _BPW + ch * _CB
    one = jnp.float32(1.0)

    def group_body(g, carry):
      pg = g * 16
      pvec = pg + iota
      bbv = lax.div(pvec, _K)
      nr0 = bbv * (_J * _K) + (pvec - bbv * _K)

      accp = jnp.zeros((16,), jnp.float32)
      accn = [jnp.zeros((16,), jnp.float32) for _ in range(_J)]
      for i in range(16):
        p = pg + i
        bb = lax.div(p, _K)
        kk = p - bb * _K
        nbase = bb * (_J * _K) + kk
        vv = [vbuf[p, pl.ds(16 * t, 16)] for t in range(4)]
        uu = [ubuf[p, pl.ds(16 * t, 16)] for t in range(4)]
        sp = (vv[0] * uu[0] + vv[1] * uu[1]
              + vv[2] * uu[2] + vv[3] * uu[3])
        accp = jnp.where(iota == i, jnp.sum(sp), accp)
        for j in range(_J):
          nr = nbase + j * _K
          nn = [nbuf[nr, pl.ds(16 * t, 16)] for t in range(4)]
          sn = (vv[0] * nn[0] + vv[1] * nn[1]
                + vv[2] * nn[2] + vv[3] * nn[3])
          accn[j] = jnp.where(iota == i, jnp.sum(sn), accn[j])

      posbuf[pl.ds(pg, 16)] = one / (one + jnp.exp(-accp))
      for j in range(_J):
        plsc.store_scatter(noutbuf, [nr0 + j * _K],
                           one / (one + jnp.exp(accn[j])))
      return carry

    lax.fori_loop(0, _GROUPS, group_body, 0)
    pltpu.sync_copy(posbuf, pos_hbm.at[pl.ds(b0 * _K, _PAIRS)])
    pltpu.sync_copy(noutbuf, nout_hbm.at[pl.ds(b0 * (_J * _K), _NROWS)])

  stage_idx(sets[0], 0)
  wait_idx(sets[0], 0)
  fire_gathers(sets[0])
  stage_idx(sets[1], 1)

  def body(i, carry):
    e = 2 * i
    o = e + 1
    wait_idx(sets[1], o)
    fire_gathers(sets[1])

    wait_gathers(sets[0])

    @pl.when(e + 2 < _NCH)
    def _():
      stage_idx(sets[0], e + 2)

    compute(sets[0], e)

    @pl.when(e + 2 < _NCH)
    def _():
      wait_idx(sets[0], e + 2)
      fire_gathers(sets[0])

    wait_gathers(sets[1])

    @pl.when(o + 2 < _NCH)
    def _():
      stage_idx(sets[1], o + 2)

    compute(sets[1], o)
    return carry

  lax.fori_loop(0, _NCH // 2, body, 0)


def kernel(c, o, neg, vEmb, uEmb):
  c_f = c.reshape(-1).astype(jnp.int32)
  o_f = o.reshape(-1).astype(jnp.int32)
  neg_f = neg.reshape(_B, _J * _K).astype(jnp.int32)
  pos, nout = _sgns(c_f, o_f, neg_f, vEmb, uEmb)
  return pos.reshape(_B, _K), nout.reshape(_B, _J, _K)


# speedup vs baseline: 1.7275x; 1.0096x over previous
import functools

import jax
import jax.numpy as jnp
from jax import lax
from jax.experimental import pallas as pl
from jax.experimental.pallas import tpu as pltpu
from jax.experimental.pallas import tpu_sc as plsc

_D = 64
_B = 4096
_K = 20
_J = 5
_NC, _NS = 2, 16
_NW = _NC * _NS
_BPW = _B // _NW
_CB = 4
_NCH = _BPW // _CB
_PAIRS = _CB * _K
_GROUPS = _PAIRS // 16
_NROWS = _CB * _J * _K

_mesh = plsc.VectorSubcoreMesh(core_axis_name="c", subcore_axis_name="s")


def _chunk_buffers():
  return [
      pltpu.VMEM((_PAIRS,), jnp.int32),
      pltpu.VMEM((_PAIRS,), jnp.int32),
      pltpu.VMEM((_CB, _J * _K), jnp.int32),
      pltpu.VMEM((_PAIRS, _D), jnp.float32),
      pltpu.VMEM((_PAIRS, _D), jnp.float32),
      pltpu.VMEM((_NROWS, _D), jnp.float32),
      pltpu.VMEM((_PAIRS,), jnp.float32),
      pltpu.VMEM((_NROWS,), jnp.float32),
      pltpu.SemaphoreType.DMA,
      pltpu.SemaphoreType.DMA,
      pltpu.SemaphoreType.DMA,
  ]


@functools.partial(
    pl.kernel,
    out_type=(
        jax.ShapeDtypeStruct((_B * _K,), jnp.float32),
        jax.ShapeDtypeStruct((_B * _J * _K,), jnp.float32),
    ),
    mesh=_mesh,
    compiler_params=pltpu.CompilerParams(
        needs_layout_passes=False, use_tc_tiling_on_sc=False),
    scratch_types=_chunk_buffers() + _chunk_buffers(),
)
def _sgns(c_hbm, o_hbm, neg_hbm, v_hbm, u_hbm, pos_hbm, nout_hbm, *bufs):
  wid = lax.axis_index("s") * _NC + lax.axis_index("c")
  iota = lax.broadcasted_iota(jnp.int32, (16,), 0)
  sets = (bufs[:11], bufs[11:22])

  def idx_slices(ch):
    b0 = wid * _BPW + ch * _CB
    p0 = b0 * _K
    return (c_hbm.at[pl.ds(p0, _PAIRS)], o_hbm.at[pl.ds(p0, _PAIRS)],
            neg_hbm.at[pl.ds(b0, _CB)])

  def stage_idx(s, ch):
    cidx, oidx, nidx, sem = s[0], s[1], s[2], s[8]
    csl, osl, nsl = idx_slices(ch)
    pltpu.async_copy(csl, cidx, sem)
    pltpu.async_copy(osl, oidx, sem)
    pltpu.async_copy(nsl, nidx, sem)

  def wait_idx(s, ch):
    cidx, oidx, nidx, sem = s[0], s[1], s[2], s[8]
    csl, osl, nsl = idx_slices(ch)
    pltpu.make_async_copy(csl, cidx, sem).wait()
    pltpu.make_async_copy(osl, oidx, sem).wait()
    pltpu.make_async_copy(nsl, nidx, sem).wait()

  def fire_gathers(s):
    cidx, oidx, nidx, vbuf, ubuf, nbuf, sem = (
        s[0], s[1], s[2], s[3], s[4], s[5], s[9])
    pltpu.async_copy(v_hbm.at[cidx], vbuf, sem)
    pltpu.async_copy(u_hbm.at[oidx], ubuf, sem)
    for i in range(_CB):
      pltpu.async_copy(u_hbm.at[nidx.at[i]],
                       nbuf.at[pl.ds(i * _J * _K, _J * _K)], sem)

  def wait_gathers(s):
    cidx, oidx, nidx, vbuf, ubuf, nbuf, sem = (
        s[0], s[1], s[2], s[3], s[4], s[5], s[9])
    pltpu.make_async_copy(v_hbm.at[cidx], vbuf, sem).wait()
    pltpu.make_async_copy(u_hbm.at[oidx], ubuf, sem).wait()
    for i in range(_CB):
      pltpu.make_async_copy(u_hbm.at[nidx.at[i]],
                            nbuf.at[pl.ds(i * _J * _K, _J * _K)], sem).wait()

  def out_slices(ch):
    b0 = wid * _BPW + ch * _CB
    return (pos_hbm.at[pl.ds(b0 * _K, _PAIRS)],
            nout_hbm.at[pl.ds(b0 * (_J * _K), _NROWS)])

  def drain_out(s, ch):
    posbuf, noutbuf, osem = s[6], s[7], s[10]
    psl, nsl = out_slices(ch)
    pltpu.make_async_copy(posbuf, psl, osem).wait()
    pltpu.make_async_copy(noutbuf, nsl, osem).wait()

  def compute(s, ch):
    vbuf, ubuf, nbuf, posbuf, noutbuf = s[3], s[4], s[5], s[6], s[7]
    b0 = wid * _BPW + ch * _CB
    one = jnp.float32(1.0)

    @pl.when(ch >= 2)
    def _():
      drain_out(s, ch - 2)

    def group_body(g, carry):
      pg = g * 16
      pvec = pg + iota
      bbv = lax.div(pvec, _K)
      nr0 = bbv * (_J * _K) + (pvec - bbv * _K)

      accp = jnp.zeros((16,), jnp.float32)
      accn = [jnp.zeros((16,), jnp.float32) for _ in range(_J)]
      for i in range(16):
        p = pg + i
        bb = lax.div(p, _K)
        kk = p - bb * _K
        nbase = bb * (_J * _K) + kk
        vv = [vbuf[p, pl.ds(16 * t, 16)] for t in range(4)]
        uu = [ubuf[p, pl.ds(16 * t, 16)] for t in range(4)]
        sp = (vv[0] * uu[0] + vv[1] * uu[1]
              + vv[2] * uu[2] + vv[3] * uu[3])
        accp = jnp.where(iota == i, jnp.sum(sp), accp)
        for j in range(_J):
          nr = nbase + j * _K
          nn = [nbuf[nr, pl.ds(16 * t, 16)] for t in range(4)]
          sn = (vv[0] * nn[0] + vv[1] * nn[1]
                + vv[2] * nn[2] + vv[3] * nn[3])
          accn[j] = jnp.where(iota == i, jnp.sum(sn), accn[j])

      posbuf[pl.ds(pg, 16)] = one / (one + jnp.exp(-accp))
      for j in range(_J):
        plsc.store_scatter(noutbuf, [nr0 + j * _K],
                           one / (one + jnp.exp(accn[j])))
      return carry

    lax.fori_loop(0, _GROUPS, group_body, 0)
    psl, nsl = out_slices(ch)
    pltpu.async_copy(posbuf, psl, s[10])
    pltpu.async_copy(noutbuf, nsl, s[10])

  stage_idx(sets[0], 0)
  wait_idx(sets[0], 0)
  fire_gathers(sets[0])
  stage_idx(sets[1], 1)

  def body(i, carry):
    e = 2 * i
    o = e + 1
    wait_idx(sets[1], o)
    fire_gathers(sets[1])

    wait_gathers(sets[0])

    @pl.when(e + 2 < _NCH)
    def _():
      stage_idx(sets[0], e + 2)

    compute(sets[0], e)

    @pl.when(e + 2 < _NCH)
    def _():
      wait_idx(sets[0], e + 2)
      fire_gathers(sets[0])

    wait_gathers(sets[1])

    @pl.when(o + 2 < _NCH)
    def _():
      stage_idx(sets[1], o + 2)

    compute(sets[1], o)
    return carry

  lax.fori_loop(0, _NCH // 2, body, 0)
  drain_out(sets[0], _NCH - 2)
  drain_out(sets[1], _NCH - 1)


def kernel(c, o, neg, vEmb, uEmb):
  c_f = c.reshape(-1).astype(jnp.int32)
  o_f = o.reshape(-1).astype(jnp.int32)
  neg_f = neg.reshape(_B, _J * _K).astype(jnp.int32)
  pos, nout = _sgns(c_f, o_f, neg_f, vEmb, uEmb)
  return pos.reshape(_B, _K), nout.reshape(_B, _J, _K)
